# baseline (speedup 1.0000x reference)
import dataclasses

import jax
import jax.numpy as jnp
from jax import lax
from jax.experimental import pallas as pl
from jax.experimental.pallas import tpu as pltpu
from jax.experimental.pallas import tpu_sc as plsc

N = 10000
D = 128
NROWS = 10240
TRASH = N
NC, NS = 2, 16
NW = NC * NS
B = 512
BS = 256
BD = 128
RPS = NROWS // NS
f32 = jnp.float32


def _z():
    return jnp.int32(0)


_mesh = plsc.VectorSubcoreMesh(core_axis_name="c", subcore_axis_name="s")

_sc_params = pltpu.CompilerParams()
if "needs_layout_passes" in pltpu.CompilerParams.__dataclass_fields__:
    _sc_params = dataclasses.replace(_sc_params, needs_layout_passes=False)


def _sc_degree(dst_pad, ep):
    epw = ep // NW
    nb = epw // BD

    @pl.kernel(out_type=jax.ShapeDtypeStruct((NW, NROWS), f32),
               mesh=_mesh,
               compiler_params=_sc_params,
               scratch_types=[pltpu.VMEM((epw,), jnp.int32),
                              pltpu.VMEM((NROWS,), f32),
                              pltpu.SemaphoreType.DMA])
    def deg_kernel(dst_hbm, out_hbm, idx_v, deg_v, sem):
        cid = lax.axis_index("c").astype(jnp.int32)
        sid = lax.axis_index("s").astype(jnp.int32)
        wid = cid * jnp.int32(NS) + sid
        zeros16 = jnp.zeros((16,), f32)
        ones16 = jnp.ones((16,), f32)
        base = wid * jnp.int32(epw)

        idx_copy = pltpu.make_async_copy(dst_hbm.at[pl.ds(base, epw)],
                                         idx_v, sem)
        idx_copy.start()

        @pl.loop(jnp.int32(0), jnp.int32(NROWS // 16))
        def _(i):
            i = jnp.asarray(i, jnp.int32)
            deg_v[pl.ds(i * jnp.int32(16), 16)] = zeros16

        idx_copy.wait()

        @pl.loop(jnp.int32(0), jnp.int32(nb))
        def _(b):
            b = jnp.asarray(b, jnp.int32)
            boff = b * jnp.int32(BD)
            for j in range(BD // 16):
                idx = idx_v[pl.ds(boff + jnp.int32(j * 16), 16)]
                plsc.addupdate_scatter(deg_v, [idx], ones16)

        pltpu.sync_copy(deg_v, out_hbm.at[wid])

    return deg_kernel(dst_pad)


def _sc_edge_agg(hs, src2, dst2, ep):
    epw = ep // NW
    H = 4
    eph = epw // H
    nbh = eph // BS
    assert nbh >= 1 and eph % BS == 0 and RPS % BD == 0

    @pl.kernel(out_type=jax.ShapeDtypeStruct((NC, NROWS, D), f32),
               mesh=_mesh,
               scratch_types=[pltpu.VMEM((eph,), jnp.int32),
                              pltpu.VMEM((epw // BD, BD), jnp.int32),
                              pltpu.VMEM((BS, D), f32),
                              pltpu.VMEM_SHARED((NROWS, D), f32),
                              pltpu.SemaphoreType.DMA,
                              pltpu.SemaphoreType.DMA])
    def agg_kernel(hs_hbm, src_hbm, dst_hbm, out_hbm,
                   src_v, dst_v, rows_v, acc_sh, isem, gsem):
        cid = lax.axis_index("c").astype(jnp.int32)
        sid = lax.axis_index("s").astype(jnp.int32)
        wid = cid * jnp.int32(NS) + sid
        r0 = sid * jnp.int32(RPS)
        z16 = jnp.zeros((16,), f32)

        dst_cp = pltpu.make_async_copy(dst_hbm.at[wid], dst_v, isem)
        dst_cp.start()

        @pl.loop(jnp.int32(0), jnp.int32(BD))
        def _(r):
            r = jnp.asarray(r, jnp.int32)
            for c in range(D // 16):
                rows_v[r, pl.ds(jnp.int32(c * 16), 16)] = z16

        for k in range(RPS // BD):
            pltpu.sync_copy(rows_v.at[pl.ds(jnp.int32(0), BD)],
                            acc_sh.at[pl.ds(r0 + jnp.int32(k * BD), BD)])
        dst_cp.wait()
        plsc.subcore_barrier()

        for h in range(H):
            src_cp = pltpu.make_async_copy(
                src_hbm.at[wid, pl.ds(jnp.int32(h * eph), eph)],
                src_v, isem)
            src_cp.start()
            src_cp.wait()

            @pl.loop(jnp.int32(0), jnp.int32(nbh))
            def _(j):
                j = jnp.asarray(j, jnp.int32)
                idx = src_v.at[pl.ds(j * jnp.int32(BS), BS)]
                pltpu.make_async_copy(hs_hbm.at[idx], rows_v, gsem).start()
                pltpu.make_async_copy(hs_hbm.at[idx], rows_v, gsem).wait()
                jb = (jnp.int32(h * nbh) + j) * jnp.int32(BS // BD)
                for k in range(BS // BD):
                    pltpu.sync_copy(
                        rows_v.at[pl.ds(jnp.int32(k * BD), BD)],
                        acc_sh.at[dst_v.at[jb + jnp.int32(k)]],
                        add=True)

        plsc.subcore_barrier()
        pltpu.sync_copy(acc_sh.at[pl.ds(r0, RPS)],
                        out_hbm.at[cid, pl.ds(r0, RPS)])

    return agg_kernel(hs, src2, dst2)


def _tc_prep(x_pad, W, cnt):
    RB = 1024

    def body(x_ref, w_ref, cnt_ref, hs_ref, hself_ref):
        h = lax.dot(x_ref[...], w_ref[...],
                    precision=lax.Precision.HIGHEST)
        deg = jnp.sum(cnt_ref[...], axis=0)[:, None] + 1.0
        hs_ref[...] = h * lax.rsqrt(deg)
        hself_ref[...] = h / deg

    return pl.pallas_call(
        body,
        grid=(NROWS // RB,),
        in_specs=[pl.BlockSpec((RB, D), lambda i: (i, _z())),
                  pl.BlockSpec((D, D), lambda i: (_z(), _z())),
                  pl.BlockSpec((NW, RB), lambda i: (_z(), i))],
        out_specs=[pl.BlockSpec((RB, D), lambda i: (i, _z())),
                   pl.BlockSpec((RB, D), lambda i: (i, _z()))],
        out_shape=[jax.ShapeDtypeStruct((NROWS, D), f32),
                   jax.ShapeDtypeStruct((NROWS, D), f32)],
    )(x_pad, W, cnt)


def _tc_final(accs, cnt, hself):
    RB = 1024

    def body(acc_ref, cnt_ref, hself_ref, out_ref):
        deg = jnp.sum(cnt_ref[...], axis=0)[:, None] + 1.0
        out_ref[...] = ((acc_ref[0] + acc_ref[1]) * lax.rsqrt(deg)
                        + hself_ref[...])

    return pl.pallas_call(
        body,
        grid=(NROWS // RB,),
        in_specs=[pl.BlockSpec((NC, RB, D), lambda i: (_z(), i, _z())),
                  pl.BlockSpec((NW, RB), lambda i: (_z(), i)),
                  pl.BlockSpec((RB, D), lambda i: (i, _z()))],
        out_specs=pl.BlockSpec((RB, D), lambda i: (i, _z())),
        out_shape=jax.ShapeDtypeStruct((NROWS, D), f32),
    )(accs, cnt, hself)


def kernel(x, edge_index, W):
    src = edge_index[0].astype(jnp.int32)
    dst = edge_index[1].astype(jnp.int32)
    e = src.shape[0]
    chunk = NW * B
    ep = ((e + chunk - 1) // chunk) * chunk
    pad = ep - e
    if pad:
        src = jnp.concatenate([src, jnp.full((pad,), TRASH, jnp.int32)])
        dst = jnp.concatenate([dst, jnp.full((pad,), TRASH, jnp.int32)])
    x_pad = jnp.pad(x.astype(f32), ((0, NROWS - N), (0, 0)))

    chunk_d = NW * BD
    epd = ((e + chunk_d - 1) // chunk_d) * chunk_d
    if epd <= ep:
        dst_d = dst[:epd] if epd < ep else dst
    else:
        dst_d = jnp.concatenate(
            [dst, jnp.full((epd - ep,), TRASH, jnp.int32)])

    cnt = _sc_degree(dst_d, epd)
    hs, hself = _tc_prep(x_pad, W.astype(f32), cnt)
    epw = ep // NW
    accs = _sc_edge_agg(hs, src.reshape(NW, epw),
                        dst.reshape(NW, epw // BD, BD), ep)
    out = _tc_final(accs, cnt, hself)
    return out[:N]


# speedup vs baseline: 1.0456x; 1.0456x over previous
import dataclasses

import jax
import jax.numpy as jnp
from jax import lax
from jax.experimental import pallas as pl
from jax.experimental.pallas import tpu as pltpu
from jax.experimental.pallas import tpu_sc as plsc

N = 10000
D = 128
NROWS = 10240
TRASH = N
NC, NS = 2, 16
NW = NC * NS
B = 512
BS = 256
BD = 128
RPS = NROWS // NS
f32 = jnp.float32


def _z():
    return jnp.int32(0)


_mesh = plsc.VectorSubcoreMesh(core_axis_name="c", subcore_axis_name="s")

_sc_params = pltpu.CompilerParams()
if "needs_layout_passes" in pltpu.CompilerParams.__dataclass_fields__:
    _sc_params = dataclasses.replace(_sc_params, needs_layout_passes=False)


def _sc_degree(dst_pad, ep):
    epw = ep // NW
    nb = epw // BD

    @pl.kernel(out_type=jax.ShapeDtypeStruct((NW, NROWS), f32),
               mesh=_mesh,
               compiler_params=_sc_params,
               scratch_types=[pltpu.VMEM((epw,), jnp.int32),
                              pltpu.VMEM((NROWS,), f32),
                              pltpu.SemaphoreType.DMA])
    def deg_kernel(dst_hbm, out_hbm, idx_v, deg_v, sem):
        cid = lax.axis_index("c").astype(jnp.int32)
        sid = lax.axis_index("s").astype(jnp.int32)
        wid = cid * jnp.int32(NS) + sid
        zeros16 = jnp.zeros((16,), f32)
        ones16 = jnp.ones((16,), f32)
        base = wid * jnp.int32(epw)

        idx_copy = pltpu.make_async_copy(dst_hbm.at[pl.ds(base, epw)],
                                         idx_v, sem)
        idx_copy.start()

        @pl.loop(jnp.int32(0), jnp.int32(NROWS // 16))
        def _(i):
            i = jnp.asarray(i, jnp.int32)
            deg_v[pl.ds(i * jnp.int32(16), 16)] = zeros16

        idx_copy.wait()

        @pl.loop(jnp.int32(0), jnp.int32(nb))
        def _(b):
            b = jnp.asarray(b, jnp.int32)
            boff = b * jnp.int32(BD)
            for j in range(BD // 16):
                idx = idx_v[pl.ds(boff + jnp.int32(j * 16), 16)]
                plsc.addupdate_scatter(deg_v, [idx], ones16)

        pltpu.sync_copy(deg_v, out_hbm.at[wid])

    return deg_kernel(dst_pad)


def _sc_edge_agg(hs, src2, dst2, ep):
    epw = ep // NW
    H = 2
    eph = epw // H
    nbh = eph // BD
    npair = nbh // 2
    assert npair >= 1 and nbh % 2 == 0 and RPS % BD == 0

    @pl.kernel(out_type=jax.ShapeDtypeStruct((NC, NROWS, D), f32),
               mesh=_mesh,
               scratch_types=[pltpu.VMEM((eph,), jnp.int32),
                              pltpu.VMEM((nbh, BD), jnp.int32),
                              pltpu.VMEM((2, BD, D), f32),
                              pltpu.VMEM_SHARED((NROWS, D), f32),
                              pltpu.SemaphoreType.DMA,
                              pltpu.SemaphoreType.DMA,
                              pltpu.SemaphoreType.DMA])
    def agg_kernel(hs_hbm, src_hbm, dst_hbm, out_hbm,
                   src_v, dst_v, rows_v, acc_sh, isem, gsem0, gsem1):
        cid = lax.axis_index("c").astype(jnp.int32)
        sid = lax.axis_index("s").astype(jnp.int32)
        wid = cid * jnp.int32(NS) + sid
        r0 = sid * jnp.int32(RPS)
        z16 = jnp.zeros((16,), f32)

        def gath(batch, buf, sem):
            idx = src_v.at[pl.ds(batch * jnp.int32(BD), BD)]
            return pltpu.make_async_copy(hs_hbm.at[idx],
                                         rows_v.at[jnp.int32(buf)], sem)

        def scat(batch, buf):
            pltpu.sync_copy(rows_v.at[jnp.int32(buf)],
                            acc_sh.at[dst_v.at[batch]], add=True)

        @pl.loop(jnp.int32(0), jnp.int32(BD))
        def _(r):
            r = jnp.asarray(r, jnp.int32)
            for c in range(D // 16):
                rows_v[jnp.int32(0), r, pl.ds(jnp.int32(c * 16), 16)] = z16

        for k in range(RPS // BD):
            pltpu.sync_copy(rows_v.at[jnp.int32(0)],
                            acc_sh.at[pl.ds(r0 + jnp.int32(k * BD), BD)])
        plsc.subcore_barrier()

        for h in range(H):
            src_cp = pltpu.make_async_copy(
                src_hbm.at[wid, pl.ds(jnp.int32(h * eph), eph)],
                src_v, isem)
            dst_cp = pltpu.make_async_copy(
                dst_hbm.at[wid, pl.ds(jnp.int32(h * nbh), nbh)],
                dst_v, isem)
            src_cp.start()
            dst_cp.start()
            src_cp.wait()
            dst_cp.wait()

            gath(jnp.int32(0), 0, gsem0).start()

            @pl.loop(jnp.int32(0), jnp.int32(npair - 1))
            def _(p):
                j0 = jnp.asarray(p, jnp.int32) * jnp.int32(2)
                gath(j0, 0, gsem0).wait()
                gath(j0 + jnp.int32(1), 1, gsem1).start()
                scat(j0, 0)
                gath(j0 + jnp.int32(1), 1, gsem1).wait()
                gath(j0 + jnp.int32(2), 0, gsem0).start()
                scat(j0 + jnp.int32(1), 1)

            j0 = jnp.int32(2 * (npair - 1))
            gath(j0, 0, gsem0).wait()
            gath(j0 + jnp.int32(1), 1, gsem1).start()
            scat(j0, 0)
            gath(j0 + jnp.int32(1), 1, gsem1).wait()
            scat(j0 + jnp.int32(1), 1)

        plsc.subcore_barrier()
        pltpu.sync_copy(acc_sh.at[pl.ds(r0, RPS)],
                        out_hbm.at[cid, pl.ds(r0, RPS)])

    return agg_kernel(hs, src2, dst2)


def _tc_prep(x_pad, W, cnt):
    RB = 1024

    def body(x_ref, w_ref, cnt_ref, hs_ref, hself_ref):
        h = lax.dot(x_ref[...], w_ref[...],
                    precision=lax.Precision.HIGHEST)
        deg = jnp.sum(cnt_ref[...], axis=0)[:, None] + 1.0
        hs_ref[...] = h * lax.rsqrt(deg)
        hself_ref[...] = h / deg

    return pl.pallas_call(
        body,
        grid=(NROWS // RB,),
        in_specs=[pl.BlockSpec((RB, D), lambda i: (i, _z())),
                  pl.BlockSpec((D, D), lambda i: (_z(), _z())),
                  pl.BlockSpec((NW, RB), lambda i: (_z(), i))],
        out_specs=[pl.BlockSpec((RB, D), lambda i: (i, _z())),
                   pl.BlockSpec((RB, D), lambda i: (i, _z()))],
        out_shape=[jax.ShapeDtypeStruct((NROWS, D), f32),
                   jax.ShapeDtypeStruct((NROWS, D), f32)],
    )(x_pad, W, cnt)


def _tc_final(accs, cnt, hself):
    RB = 1024

    def body(acc_ref, cnt_ref, hself_ref, out_ref):
        deg = jnp.sum(cnt_ref[...], axis=0)[:, None] + 1.0
        out_ref[...] = ((acc_ref[0] + acc_ref[1]) * lax.rsqrt(deg)
                        + hself_ref[...])

    return pl.pallas_call(
        body,
        grid=(NROWS // RB,),
        in_specs=[pl.BlockSpec((NC, RB, D), lambda i: (_z(), i, _z())),
                  pl.BlockSpec((NW, RB), lambda i: (_z(), i)),
                  pl.BlockSpec((RB, D), lambda i: (i, _z()))],
        out_specs=pl.BlockSpec((RB, D), lambda i: (i, _z())),
        out_shape=jax.ShapeDtypeStruct((NROWS, D), f32),
    )(accs, cnt, hself)


def kernel(x, edge_index, W):
    src = edge_index[0].astype(jnp.int32)
    dst = edge_index[1].astype(jnp.int32)
    e = src.shape[0]
    chunk = NW * B
    ep = ((e + chunk - 1) // chunk) * chunk
    pad = ep - e
    if pad:
        src = jnp.concatenate([src, jnp.full((pad,), TRASH, jnp.int32)])
        dst = jnp.concatenate([dst, jnp.full((pad,), TRASH, jnp.int32)])
    x_pad = jnp.pad(x.astype(f32), ((0, NROWS - N), (0, 0)))

    chunk_d = NW * BD
    epd = ((e + chunk_d - 1) // chunk_d) * chunk_d
    if epd <= ep:
        dst_d = dst[:epd] if epd < ep else dst
    else:
        dst_d = jnp.concatenate(
            [dst, jnp.full((epd - ep,), TRASH, jnp.int32)])

    cnt = _sc_degree(dst_d, epd)
    hs, hself = _tc_prep(x_pad, W.astype(f32), cnt)
    epw = ep // NW
    accs = _sc_edge_agg(hs, src.reshape(NW, epw),
                        dst.reshape(NW, epw // BD, BD), ep)
    out = _tc_final(accs, cnt, hself)
    return out[:N]
    out = _tc_final(accs, cnt, hself)
    return out[:N]
